# baseline (speedup 1.0000x reference)
import functools

import jax
import jax.numpy as jnp
from jax import lax
from jax.experimental import pallas as pl
from jax.experimental.pallas import tpu as pltpu
from jax.experimental.pallas import tpu_sc as plsc

B = 128
T = 1024
C = 192
ROWS = B * T

NUM_WORKERS = 32
ROWS_PER_WORKER = ROWS // NUM_WORKERS
CHUNK = 128
CHUNKS_PER_WORKER = ROWS_PER_WORKER // CHUNK


def _sc_gather(x2d, abs_idx):
    mesh = plsc.VectorSubcoreMesh(core_axis_name="c", subcore_axis_name="s")

    @functools.partial(
        pl.kernel,
        mesh=mesh,
        compiler_params=pltpu.CompilerParams(use_tc_tiling_on_sc=False),
        out_type=jax.ShapeDtypeStruct((ROWS, C), jnp.float32),
        scratch_types=[
            pltpu.VMEM((CHUNK,), jnp.int32),
            pltpu.VMEM((CHUNK, C), jnp.float32),
            pltpu.SemaphoreType.DMA,
        ],
    )
    def k(x_hbm, idx_hbm, out_hbm, idx_v, rows_v, sem):
        wid = lax.axis_index("s") * 2 + lax.axis_index("c")
        w_base = wid * ROWS_PER_WORKER

        def body(i, carry):
            base = w_base + i * CHUNK
            pltpu.sync_copy(idx_hbm.at[pl.ds(base, CHUNK)], idx_v)
            pltpu.async_copy(x_hbm.at[idx_v], rows_v, sem).wait()
            pltpu.sync_copy(rows_v, out_hbm.at[pl.ds(base, CHUNK)])
            return carry

        lax.fori_loop(0, CHUNKS_PER_WORKER, body, 0)

    return k(x2d, abs_idx)


def kernel(x, forward_shuffle_idx):
    idx = forward_shuffle_idx.astype(jnp.int32)
    abs_idx = (jnp.arange(B, dtype=jnp.int32)[:, None] * T + idx[None, :]).reshape(-1)
    out2d = _sc_gather(x.reshape(ROWS, C), abs_idx)
    return out2d.reshape(B, T, C)


# speedup vs baseline: 1.0183x; 1.0183x over previous
import functools

import jax
import jax.numpy as jnp
from jax import lax
from jax.experimental import pallas as pl
from jax.experimental.pallas import tpu as pltpu
from jax.experimental.pallas import tpu_sc as plsc

B = 128
T = 1024
C = 192

NUM_WORKERS = 32
BATCHES_PER_WORKER = B // NUM_WORKERS
CHUNK = 128
CHUNKS_PER_BATCH = T // CHUNK
CHUNKS_PER_WORKER = BATCHES_PER_WORKER * CHUNKS_PER_BATCH


def _sc_gather(x, idx):
    mesh = plsc.VectorSubcoreMesh(core_axis_name="c", subcore_axis_name="s")

    @functools.partial(
        pl.kernel,
        mesh=mesh,
        compiler_params=pltpu.CompilerParams(use_tc_tiling_on_sc=False),
        out_type=jax.ShapeDtypeStruct((B, T, C), jnp.float32),
        scratch_types=[
            pltpu.VMEM((T,), jnp.int32),
            pltpu.VMEM((CHUNK, C), jnp.float32),
            pltpu.SemaphoreType.DMA,
        ],
    )
    def k(x_hbm, idx_hbm, out_hbm, idx_v, rows_v, sem):
        wid = lax.axis_index("s") * 2 + lax.axis_index("c")
        pltpu.sync_copy(idx_hbm, idx_v)

        def body(i, carry):
            b = wid * BATCHES_PER_WORKER + i // CHUNKS_PER_BATCH
            t0 = (i % CHUNKS_PER_BATCH) * CHUNK
            pltpu.async_copy(
                x_hbm.at[b].at[idx_v.at[pl.ds(t0, CHUNK)]], rows_v, sem
            ).wait()
            pltpu.sync_copy(rows_v, out_hbm.at[b].at[pl.ds(t0, CHUNK)])
            return carry

        lax.fori_loop(0, CHUNKS_PER_WORKER, body, 0)

    return k(x, idx)


def kernel(x, forward_shuffle_idx):
    return _sc_gather(x, forward_shuffle_idx.astype(jnp.int32))
